# baseline (speedup 1.0000x reference)
import functools

import jax
import jax.numpy as jnp
from jax import lax
from jax.experimental import pallas as pl

_SMOOTH = 0.001
_NEG_BIG = -1e30
_BIG = 1e30


def _corners4(x):
    x01 = jnp.roll(x, -1, axis=1)
    x10 = jnp.roll(x, -1, axis=0)
    x11 = jnp.roll(x10, -1, axis=1)
    return [x, x01, x10, x11]


def _dotc(a, b):
    return lax.dot_general(a, b, (((1,), (1,)), ((), ())),
                           preferred_element_type=jnp.float32)


def _hist_kernel(p0_ref, p1_ref, l0_ref, l1_ref, out_ref, *,
                 valid_r, valid_c, seg_w, n_seg):
    zp = pl.program_id(1)
    f32 = jnp.float32
    bf16 = jnp.bfloat16

    s0 = jax.nn.sigmoid(p0_ref[0, 0])
    s1 = jax.nn.sigmoid(p1_ref[0, 0])
    R, C = s0.shape
    N = R * C

    cp = _corners4(s0) + _corners4(s1)

    la = _corners4(l0_ref[0, 0])
    lb_ = _corners4(l1_ref[0, 0])
    llo = la[0] + 2.0 * la[1] + 4.0 * la[2] + 8.0 * la[3]
    lhi = lb_[0] + 2.0 * lb_[1] + 4.0 * lb_[2] + 8.0 * lb_[3]

    i16 = lax.broadcasted_iota(jnp.int32, (16, N), 0).astype(bf16)
    zero16 = jnp.zeros((16, N), bf16)

    pw = jnp.zeros((R, C), f32)
    hp = jnp.zeros((16, 16), f32)

    for _ in range(8):
        nzb = [c > 0 for c in cp]
        nzf = [jnp.where(b, f32(1.0), f32(0.0)) for b in nzb]
        masked = [jnp.where(b, c, f32(_BIG)) for b, c in zip(nzb, cp)]
        sw = jnp.minimum(
            jnp.minimum(jnp.minimum(masked[0], masked[1]),
                        jnp.minimum(masked[2], masked[3])),
            jnp.minimum(jnp.minimum(masked[4], masked[5]),
                        jnp.minimum(masked[6], masked[7])))
        sw = jnp.where(sw < f32(1e29), sw, f32(0.0))

        lo = nzf[0] + 2.0 * nzf[1] + 4.0 * nzf[2] + 8.0 * nzf[3]
        hi = nzf[4] + 2.0 * nzf[5] + 4.0 * nzf[6] + 8.0 * nzf[7]

        match = (lo == llo) & (hi == lhi)
        pw = pw + jnp.where(match, sw, f32(0.0))

        lo_f = lo.astype(bf16).reshape(1, N)
        hi_f = hi.astype(bf16).reshape(1, N)
        sw_f = sw.astype(bf16).reshape(1, N)
        x = jnp.where(hi_f == i16, sw_f, zero16)
        y = jnp.where(lo_f == i16, bf16(1.0), bf16(0.0))
        hp = hp + _dotc(x, y)

        cp = [c - sw * f for c, f in zip(cp, nzf)]

    rowv = (lax.broadcasted_iota(jnp.int32, (R, 1), 0) < valid_r).astype(bf16)
    lane = lax.broadcasted_iota(jnp.int32, (1, C), 1)
    colv = jnp.zeros((1, C), bf16)
    for s in range(n_seg):
        seg_ok = (lane >= s * seg_w) & (lane < s * seg_w + valid_c)
        colv = colv + seg_ok.astype(bf16)
    valid_f = (rowv * colv).reshape(1, N)

    llo_f = llo.astype(bf16).reshape(1, N)
    lhi_f = lhi.astype(bf16).reshape(1, N)
    pw_f = pw.astype(bf16).reshape(1, N)
    oh_llo = jnp.where(llo_f == i16, bf16(1.0), bf16(0.0))
    oh_lhi_b = lhi_f == i16
    xv = jnp.where(oh_lhi_b, valid_f, zero16)
    xpw = jnp.where(oh_lhi_b, pw_f, zero16)
    hl = _dotc(xv, oh_llo)
    hi_ = _dotc(xpw, oh_llo)

    @pl.when(zp == 0)
    def _init():
        out_ref[...] = jnp.zeros_like(out_ref)

    out_ref[0, 0] = out_ref[0, 0] + hp
    out_ref[0, 1] = out_ref[0, 1] + hl
    out_ref[0, 2] = out_ref[0, 2] + hi_


def kernel(pred, labels, area):
    B, Z, H, W = pred.shape
    P = Z - 1
    pps = 3 if P % 3 == 0 else 1
    G = P // pps
    Rp = ((H + 2 + 7) // 8) * 8
    Cs = W + 2
    Cp = ((pps * Cs + 127) // 128) * 128

    f32 = jnp.float32

    def pack(vol, fill):
        base = jnp.full((B, Z, Rp, Cs), fill, f32)
        base = base.at[:, :, 1:H + 1, 1:W + 1].set(vol.astype(f32))
        a0 = base[:, :P].reshape(B, G, pps, Rp, Cs)
        a0 = a0.transpose(0, 1, 3, 2, 4).reshape(B, G, Rp, pps * Cs)
        a1 = base[:, 1:P + 1].reshape(B, G, pps, Rp, Cs)
        a1 = a1.transpose(0, 1, 3, 2, 4).reshape(B, G, Rp, pps * Cs)
        padw = ((0, 0), (0, 0), (0, 0), (0, Cp - pps * Cs))
        return (jnp.pad(a0, padw, constant_values=fill),
                jnp.pad(a1, padw, constant_values=fill))

    p0, p1 = pack(pred, _NEG_BIG)
    l0, l1 = pack(labels, 0.0)

    grid = (B, G)
    blk = (1, 1, Rp, Cp)
    hists = pl.pallas_call(
        functools.partial(_hist_kernel, valid_r=H + 1, valid_c=W + 1,
                          seg_w=Cs, n_seg=pps),
        grid=grid,
        in_specs=[
            pl.BlockSpec(blk, lambda b, g: (b, g, 0, 0)),
            pl.BlockSpec(blk, lambda b, g: (b, g, 0, 0)),
            pl.BlockSpec(blk, lambda b, g: (b, g, 0, 0)),
            pl.BlockSpec(blk, lambda b, g: (b, g, 0, 0)),
        ],
        out_specs=pl.BlockSpec((1, 3, 16, 16), lambda b, g: (b, 0, 0, 0)),
        out_shape=jax.ShapeDtypeStruct((B, 3, 16, 16), f32),
    )(p0, p1, l0, l1)

    a16 = area.astype(f32).reshape(16, 16)
    denom = ((hists[:, 0] + hists[:, 1]) * a16[None]).sum(axis=(1, 2))
    num = 2.0 * (hists[:, 2] * a16[None]).sum(axis=(1, 2))
    dice = 1.0 - (num + _SMOOTH) / (denom + _SMOOTH)
    return dice.mean()


# speedup vs baseline: 2.7413x; 2.7413x over previous
import functools

import jax
import jax.numpy as jnp
from jax import lax
from jax.experimental import pallas as pl

_SMOOTH = 0.001
_NEG_BIG = -1e30
_BIG = 1e30


def _corners4_main(x, lane0):
    zero = jnp.float32(0.0)
    x10 = jnp.roll(x, -1, axis=0)
    c00 = jnp.where(lane0, zero, jnp.roll(x, 1, axis=1))
    c10 = jnp.where(lane0, zero, jnp.roll(x10, 1, axis=1))
    return [c00, x, c10, x10]


def _corners4_edge(x):
    x01 = jnp.roll(x, -1, axis=1)
    x10 = jnp.roll(x, -1, axis=0)
    x11 = jnp.roll(x10, -1, axis=1)
    return [x, x01, x10, x11]


def _dotc(a, b):
    return lax.dot_general(a, b, (((1,), (1,)), ((), ())),
                           preferred_element_type=jnp.float32)


def _hist_body(cp, lcorn, rowv_b, colv_b, sw_mask, out_ref, accumulate, zp):
    f32 = jnp.float32
    bf16 = jnp.bfloat16
    R, C = cp[0].shape
    N = R * C

    llo = lcorn[0] + 2.0 * lcorn[1] + 4.0 * lcorn[2] + 8.0 * lcorn[3]
    lhi = lcorn[4] + 2.0 * lcorn[5] + 4.0 * lcorn[6] + 8.0 * lcorn[7]

    i16 = lax.broadcasted_iota(jnp.int32, (16, N), 0).astype(bf16)
    zero16 = jnp.zeros((16, N), bf16)

    pw = jnp.zeros((R, C), f32)
    hp = jnp.zeros((16, 16), f32)

    for _ in range(8):
        nzb = [c > 0 for c in cp]
        nzf = [jnp.where(b, f32(1.0), f32(0.0)) for b in nzb]
        masked = [jnp.where(b, c, f32(_BIG)) for b, c in zip(nzb, cp)]
        sw = jnp.minimum(
            jnp.minimum(jnp.minimum(masked[0], masked[1]),
                        jnp.minimum(masked[2], masked[3])),
            jnp.minimum(jnp.minimum(masked[4], masked[5]),
                        jnp.minimum(masked[6], masked[7])))
        sw = jnp.where(sw < f32(1e29), sw, f32(0.0))
        if sw_mask is not None:
            sw = sw * sw_mask

        lo = nzf[0] + 2.0 * nzf[1] + 4.0 * nzf[2] + 8.0 * nzf[3]
        hi = nzf[4] + 2.0 * nzf[5] + 4.0 * nzf[6] + 8.0 * nzf[7]

        match = (lo == llo) & (hi == lhi)
        pw = pw + jnp.where(match, sw, f32(0.0))

        lo_f = lo.astype(bf16).reshape(1, N)
        hi_f = hi.astype(bf16).reshape(1, N)
        sw_f = sw.astype(bf16).reshape(1, N)
        x = jnp.where(hi_f == i16, sw_f, zero16)
        y = jnp.where(lo_f == i16, bf16(1.0), bf16(0.0))
        hp = hp + _dotc(x, y)

        cp = [c - sw * f for c, f in zip(cp, nzf)]

    valid_f = (rowv_b * colv_b).reshape(1, N)
    llo_f = llo.astype(bf16).reshape(1, N)
    lhi_f = lhi.astype(bf16).reshape(1, N)
    pw_f = pw.astype(bf16).reshape(1, N)
    oh_llo = jnp.where(llo_f == i16, bf16(1.0), bf16(0.0))
    oh_lhi_b = lhi_f == i16
    xv = jnp.where(oh_lhi_b, valid_f, zero16)
    xpw = jnp.where(oh_lhi_b, pw_f, zero16)
    hl = _dotc(xv, oh_llo)
    hi_ = _dotc(xpw, oh_llo)

    if accumulate:
        @pl.when(zp == 0)
        def _init():
            out_ref[...] = jnp.zeros_like(out_ref)
        out_ref[0, 0] = out_ref[0, 0] + hp
        out_ref[0, 1] = out_ref[0, 1] + hl
        out_ref[0, 2] = out_ref[0, 2] + hi_
    else:
        out_ref[0, 0] = hp
        out_ref[0, 1] = hl
        out_ref[0, 2] = hi_


def _main_kernel(p0_ref, p1_ref, l0_ref, l1_ref, out_ref, *, valid_r, valid_c):
    zp = pl.program_id(1)
    bf16 = jnp.bfloat16
    s0 = jax.nn.sigmoid(p0_ref[0, 0])
    s1 = jax.nn.sigmoid(p1_ref[0, 0])
    R, C = s0.shape
    lane0 = lax.broadcasted_iota(jnp.int32, (1, C), 1) == 0
    cp = _corners4_main(s0, lane0) + _corners4_main(s1, lane0)
    lcorn = (_corners4_main(l0_ref[0, 0], lane0) +
             _corners4_main(l1_ref[0, 0], lane0))
    rowv = (lax.broadcasted_iota(jnp.int32, (R, 1), 0) < valid_r).astype(bf16)
    colv = (lax.broadcasted_iota(jnp.int32, (1, C), 1) < valid_c).astype(bf16)
    _hist_body(cp, lcorn, rowv, colv, None, out_ref, True, zp)


def _edge_kernel(p0_ref, p1_ref, l0_ref, l1_ref, out_ref, *, valid_r, n_pair):
    f32 = jnp.float32
    bf16 = jnp.bfloat16
    s0 = jax.nn.sigmoid(p0_ref[0])
    s1 = jax.nn.sigmoid(p1_ref[0])
    R, C = s0.shape
    cp = _corners4_edge(s0) + _corners4_edge(s1)
    lcorn = _corners4_edge(l0_ref[0]) + _corners4_edge(l1_ref[0])
    lf = lax.broadcasted_iota(jnp.int32, (1, C), 1).astype(f32)
    lmod = lf - 3.0 * jnp.floor(lf * f32(0.33333334))
    in_range = lf < f32(3 * n_pair)
    rowv = (lax.broadcasted_iota(jnp.int32, (R, 1), 0) < valid_r).astype(bf16)
    colv = ((lmod == 0.0) & in_range).astype(bf16)
    sw_mask = jnp.where((lmod != 2.0) & in_range, f32(1.0), f32(0.0))
    _hist_body(cp, lcorn, rowv, colv, sw_mask, out_ref, False, None)


def kernel(pred, labels, area):
    B, Z, H, W = pred.shape
    P = Z - 1
    Rp = ((H + 2 + 7) // 8) * 8
    Cp = ((W + 127) // 128) * 128

    f32 = jnp.float32
    rpad = ((0, 0), (0, 0), (1, Rp - H - 1), (0, Cp - W))
    pm = jnp.pad(pred.astype(f32), rpad, constant_values=_NEG_BIG)
    lm = jnp.pad(labels.astype(f32), rpad, constant_values=0.0)

    grid = (B, P)
    blk = (1, 1, Rp, Cp)
    hists = pl.pallas_call(
        functools.partial(_main_kernel, valid_r=H + 1,
                          valid_c=min(W + 1, Cp)),
        grid=grid,
        in_specs=[
            pl.BlockSpec(blk, lambda b, z: (b, z, 0, 0)),
            pl.BlockSpec(blk, lambda b, z: (b, z + 1, 0, 0)),
            pl.BlockSpec(blk, lambda b, z: (b, z, 0, 0)),
            pl.BlockSpec(blk, lambda b, z: (b, z + 1, 0, 0)),
        ],
        out_specs=pl.BlockSpec((1, 3, 16, 16), lambda b, z: (b, 0, 0, 0)),
        out_shape=jax.ShapeDtypeStruct((B, 3, 16, 16), f32),
    )(pm, pm, lm, lm)

    if W % 128 == 0:
        lanes = 3 * jnp.arange(P)
        colp = jnp.pad(pred[:, :, :, W - 1].astype(f32),
                       ((0, 0), (0, 0), (1, Rp - H - 1)),
                       constant_values=_NEG_BIG)
        coll = jnp.pad(labels[:, :, :, W - 1].astype(f32),
                       ((0, 0), (0, 0), (1, Rp - H - 1)),
                       constant_values=0.0)

        def scat(col, zlo, fill):
            out = jnp.full((B, Rp, 128), fill, f32)
            return out.at[:, :, lanes].set(
                col[:, zlo:zlo + P].transpose(0, 2, 1))

        p0e = scat(colp, 0, _NEG_BIG)
        p1e = scat(colp, 1, _NEG_BIG)
        l0e = scat(coll, 0, 0.0)
        l1e = scat(coll, 1, 0.0)

        eblk = (1, Rp, 128)
        ehists = pl.pallas_call(
            functools.partial(_edge_kernel, valid_r=H + 1, n_pair=P),
            grid=(B,),
            in_specs=[pl.BlockSpec(eblk, lambda b: (b, 0, 0))] * 4,
            out_specs=pl.BlockSpec((1, 3, 16, 16), lambda b: (b, 0, 0, 0)),
            out_shape=jax.ShapeDtypeStruct((B, 3, 16, 16), f32),
        )(p0e, p1e, l0e, l1e)
        hists = hists + ehists

    a16 = area.astype(f32).reshape(16, 16)
    denom = ((hists[:, 0] + hists[:, 1]) * a16[None]).sum(axis=(1, 2))
    num = 2.0 * (hists[:, 2] * a16[None]).sum(axis=(1, 2))
    dice = 1.0 - (num + _SMOOTH) / (denom + _SMOOTH)
    return dice.mean()


# speedup vs baseline: 2.9022x; 1.0587x over previous
import functools

import jax
import jax.numpy as jnp
from jax import lax
from jax.experimental import pallas as pl

_SMOOTH = 0.001
_NEG_BIG = -1e30
_BIG = 1e30


def _corners4_main(x, lane0):
    zero = jnp.float32(0.0)
    x10 = jnp.roll(x, -1, axis=0)
    c00 = jnp.where(lane0, zero, jnp.roll(x, 1, axis=1))
    c10 = jnp.where(lane0, zero, jnp.roll(x10, 1, axis=1))
    return [c00, x, c10, x10]


def _corners4_edge(x):
    x01 = jnp.roll(x, -1, axis=1)
    x10 = jnp.roll(x, -1, axis=0)
    x11 = jnp.roll(x10, -1, axis=1)
    return [x, x01, x10, x11]


def _dotc(a, b):
    return lax.dot_general(a, b, (((1,), (1,)), ((), ())),
                           preferred_element_type=jnp.float32)


def _hist_body(cp, lcorn, rowv_b, colv_b, sw_mask, out_ref, accumulate, zp,
               static0=None):
    f32 = jnp.float32
    bf16 = jnp.bfloat16
    R, C = cp[0].shape
    N = R * C

    llo = lcorn[0] + 2.0 * lcorn[1] + 4.0 * lcorn[2] + 8.0 * lcorn[3]
    lhi = lcorn[4] + 2.0 * lcorn[5] + 4.0 * lcorn[6] + 8.0 * lcorn[7]

    i16 = lax.broadcasted_iota(jnp.int32, (16, N), 0).astype(bf16)
    zero16 = jnp.zeros((16, N), bf16)

    pw = jnp.zeros((R, C), f32)
    hp = jnp.zeros((16, 16), f32)

    for t in range(8):
        nzb = [c > 0 for c in cp]
        nzf = [jnp.where(b, f32(1.0), f32(0.0)) for b in nzb]
        masked = [jnp.where(b, c, f32(_BIG)) for b, c in zip(nzb, cp)]
        sw = jnp.minimum(
            jnp.minimum(jnp.minimum(masked[0], masked[1]),
                        jnp.minimum(masked[2], masked[3])),
            jnp.minimum(jnp.minimum(masked[4], masked[5]),
                        jnp.minimum(masked[6], masked[7])))
        sw = jnp.where(sw < f32(1e29), sw, f32(0.0))
        if sw_mask is not None:
            sw = sw * sw_mask

        lo = nzf[0] + 2.0 * nzf[1] + 4.0 * nzf[2] + 8.0 * nzf[3]
        hi = nzf[4] + 2.0 * nzf[5] + 4.0 * nzf[6] + 8.0 * nzf[7]

        match = (lo == llo) & (hi == lhi)
        pw = pw + jnp.where(match, sw, f32(0.0))

        if t == 0 and static0 is not None:
            dh, dw = static0
            t_all = jnp.sum(sw, axis=0, keepdims=True)
            t_0 = sw[0:1, :]
            t_h = sw[dh:dh + 1, :]
            t_m = t_all - t_0 - t_h
            ri = lax.broadcasted_iota(jnp.int32, (16, 1), 0)
            ci = lax.broadcasted_iota(jnp.int32, (1, 16), 1)

            def region(tr, v_mid, v_c0, v_cw):
                s0 = tr[:, 0:1]
                sw_ = tr[:, dw:dw + 1] if dw < C else jnp.zeros((1, 1), f32)
                sm = jnp.sum(tr, axis=1, keepdims=True) - s0 - sw_
                acc = sm * ((ri == v_mid) & (ci == v_mid)).astype(f32)
                acc = acc + s0 * ((ri == v_c0) & (ci == v_c0)).astype(f32)
                if dw < C:
                    acc = acc + sw_ * ((ri == v_cw) & (ci == v_cw)).astype(f32)
                return acc

            hp = hp + region(t_m, 15, 10, 5)
            hp = hp + region(t_0, 12, 8, 4)
            hp = hp + region(t_h, 3, 2, 1)
        else:
            lo_f = lo.astype(bf16).reshape(1, N)
            hi_f = hi.astype(bf16).reshape(1, N)
            sw_f = sw.astype(bf16).reshape(1, N)
            x = jnp.where(hi_f == i16, sw_f, zero16)
            y = jnp.where(lo_f == i16, bf16(1.0), bf16(0.0))
            hp = hp + _dotc(x, y)

        cp = [c - sw * f for c, f in zip(cp, nzf)]

    valid_f = (rowv_b * colv_b).reshape(1, N)
    llo_f = llo.astype(bf16).reshape(1, N)
    lhi_f = lhi.astype(bf16).reshape(1, N)
    pw_f = pw.astype(bf16).reshape(1, N)
    oh_llo = jnp.where(llo_f == i16, bf16(1.0), bf16(0.0))
    oh_lhi_b = lhi_f == i16
    xv = jnp.where(oh_lhi_b, valid_f, zero16)
    xpw = jnp.where(oh_lhi_b, pw_f, zero16)
    hl = _dotc(xv, oh_llo)
    hi_ = _dotc(xpw, oh_llo)

    if accumulate:
        @pl.when(zp == 0)
        def _init():
            out_ref[...] = jnp.zeros_like(out_ref)
        out_ref[0, 0] = out_ref[0, 0] + hp
        out_ref[0, 1] = out_ref[0, 1] + hl
        out_ref[0, 2] = out_ref[0, 2] + hi_
    else:
        out_ref[0, 0] = hp
        out_ref[0, 1] = hl
        out_ref[0, 2] = hi_


def _main_kernel(p0_ref, p1_ref, l0_ref, l1_ref, out_ref, *, valid_r, valid_c,
                 data_w):
    zp = pl.program_id(1)
    bf16 = jnp.bfloat16
    s0 = jax.nn.sigmoid(p0_ref[0, 0])
    s1 = jax.nn.sigmoid(p1_ref[0, 0])
    R, C = s0.shape
    lane0 = lax.broadcasted_iota(jnp.int32, (1, C), 1) == 0
    cp = _corners4_main(s0, lane0) + _corners4_main(s1, lane0)
    lcorn = (_corners4_main(l0_ref[0, 0], lane0) +
             _corners4_main(l1_ref[0, 0], lane0))
    rowv = (lax.broadcasted_iota(jnp.int32, (R, 1), 0) < valid_r).astype(bf16)
    colv = (lax.broadcasted_iota(jnp.int32, (1, C), 1) < valid_c).astype(bf16)
    _hist_body(cp, lcorn, rowv, colv, None, out_ref, True, zp,
               static0=(valid_r - 1, data_w))


def _edge_kernel(p0_ref, p1_ref, l0_ref, l1_ref, out_ref, *, valid_r, n_pair):
    f32 = jnp.float32
    bf16 = jnp.bfloat16
    s0 = jax.nn.sigmoid(p0_ref[0])
    s1 = jax.nn.sigmoid(p1_ref[0])
    R, C = s0.shape
    cp = _corners4_edge(s0) + _corners4_edge(s1)
    lcorn = _corners4_edge(l0_ref[0]) + _corners4_edge(l1_ref[0])
    lf = lax.broadcasted_iota(jnp.int32, (1, C), 1).astype(f32)
    lmod = lf - 3.0 * jnp.floor(lf * f32(0.33333334))
    in_range = lf < f32(3 * n_pair)
    rowv = (lax.broadcasted_iota(jnp.int32, (R, 1), 0) < valid_r).astype(bf16)
    colv = ((lmod == 0.0) & in_range).astype(bf16)
    sw_mask = jnp.where((lmod != 2.0) & in_range, f32(1.0), f32(0.0))
    _hist_body(cp, lcorn, rowv, colv, sw_mask, out_ref, False, None)


def kernel(pred, labels, area):
    B, Z, H, W = pred.shape
    P = Z - 1
    Rp = ((H + 2 + 7) // 8) * 8
    Cp = ((W + 127) // 128) * 128

    f32 = jnp.float32
    rpad = ((0, 0), (0, 0), (1, Rp - H - 1), (0, Cp - W))
    pm = jnp.pad(pred.astype(f32), rpad, constant_values=_NEG_BIG)
    lm = jnp.pad(labels.astype(f32), rpad, constant_values=0.0)

    grid = (B, P)
    blk = (1, 1, Rp, Cp)
    hists = pl.pallas_call(
        functools.partial(_main_kernel, valid_r=H + 1,
                          valid_c=min(W + 1, Cp), data_w=W),
        grid=grid,
        in_specs=[
            pl.BlockSpec(blk, lambda b, z: (b, z, 0, 0)),
            pl.BlockSpec(blk, lambda b, z: (b, z + 1, 0, 0)),
            pl.BlockSpec(blk, lambda b, z: (b, z, 0, 0)),
            pl.BlockSpec(blk, lambda b, z: (b, z + 1, 0, 0)),
        ],
        out_specs=pl.BlockSpec((1, 3, 16, 16), lambda b, z: (b, 0, 0, 0)),
        out_shape=jax.ShapeDtypeStruct((B, 3, 16, 16), f32),
    )(pm, pm, lm, lm)

    if W % 128 == 0:
        lanes = 3 * jnp.arange(P)
        colp = jnp.pad(pred[:, :, :, W - 1].astype(f32),
                       ((0, 0), (0, 0), (1, Rp - H - 1)),
                       constant_values=_NEG_BIG)
        coll = jnp.pad(labels[:, :, :, W - 1].astype(f32),
                       ((0, 0), (0, 0), (1, Rp - H - 1)),
                       constant_values=0.0)

        def scat(col, zlo, fill):
            out = jnp.full((B, Rp, 128), fill, f32)
            return out.at[:, :, lanes].set(
                col[:, zlo:zlo + P].transpose(0, 2, 1))

        p0e = scat(colp, 0, _NEG_BIG)
        p1e = scat(colp, 1, _NEG_BIG)
        l0e = scat(coll, 0, 0.0)
        l1e = scat(coll, 1, 0.0)

        eblk = (1, Rp, 128)
        ehists = pl.pallas_call(
            functools.partial(_edge_kernel, valid_r=H + 1, n_pair=P),
            grid=(B,),
            in_specs=[pl.BlockSpec(eblk, lambda b: (b, 0, 0))] * 4,
            out_specs=pl.BlockSpec((1, 3, 16, 16), lambda b: (b, 0, 0, 0)),
            out_shape=jax.ShapeDtypeStruct((B, 3, 16, 16), f32),
        )(p0e, p1e, l0e, l1e)
        hists = hists + ehists

    a16 = area.astype(f32).reshape(16, 16)
    denom = ((hists[:, 0] + hists[:, 1]) * a16[None]).sum(axis=(1, 2))
    num = 2.0 * (hists[:, 2] * a16[None]).sum(axis=(1, 2))
    dice = 1.0 - (num + _SMOOTH) / (denom + _SMOOTH)
    return dice.mean()
